# baseline (speedup 1.0000x reference)
import functools
import jax
import jax.numpy as jnp
from jax import lax
from jax.experimental import pallas as pl
from jax.experimental.pallas import tpu as pltpu
from jax.experimental.pallas import tpu_sc as plsc

BATCH = 4
SEQ = 2048
D_MODEL = 1024
L = 16
CB = 16384
UNROLL = 8

_POS_N = SEQ * D_MODEL
_NPB = _POS_N // CB


def kernel(x, pos_table):
    xf = x.reshape(BATCH * _POS_N)
    pf = pos_table.reshape(_POS_N)
    mesh = plsc.VectorSubcoreMesh(core_axis_name="core", subcore_axis_name="subcore")

    @functools.partial(
        pl.kernel,
        out_type=jax.ShapeDtypeStruct((BATCH * _POS_N,), jnp.float32),
        mesh=mesh,
        scratch_types=[],
    )
    def k(x_hbm, pos_hbm, o_hbm):
        def body(x_vmem, pos_vmem, o_vmem):
            @pl.loop(0, CB, step=L * UNROLL)
            def _(c):
                for u in range(UNROLL):
                    slc = pl.ds(c + u * L, L)
                    o_vmem.at[slc][...] = x_vmem.at[slc][...] + pos_vmem.at[slc][...]

        pltpu.emit_pipeline(
            body,
            grid=(_NPB, BATCH),
            in_specs=[
                pl.BlockSpec((CB,), lambda i, b: (b * _NPB + i,)),
                pl.BlockSpec((CB,), lambda i, b: (i,)),
            ],
            out_specs=[pl.BlockSpec((CB,), lambda i, b: (b * _NPB + i,))],
            core_axis_name=("core", "subcore"),
            dimension_semantics=(pltpu.PARALLEL, pltpu.ARBITRARY),
        )(x_hbm, pos_hbm, o_hbm)

    return k(xf, pf).reshape(BATCH, SEQ, D_MODEL)


# speedup vs baseline: 7.6548x; 7.6548x over previous
import jax
import jax.numpy as jnp
from jax.experimental import pallas as pl
from jax.experimental.pallas import tpu as pltpu

BATCH = 4
SEQ = 2048
D_MODEL = 1024
BS = 256


def _add_body(x_ref, pos_ref, o_ref):
    o_ref[...] = x_ref[...] + pos_ref[...][None, :, :]


def kernel(x, pos_table):
    n_blocks = SEQ // BS
    return pl.pallas_call(
        _add_body,
        grid=(n_blocks,),
        in_specs=[
            pl.BlockSpec((BATCH, BS, D_MODEL), lambda i: (0, i, 0)),
            pl.BlockSpec((BS, D_MODEL), lambda i: (i, 0)),
        ],
        out_specs=pl.BlockSpec((BATCH, BS, D_MODEL), lambda i: (0, i, 0)),
        out_shape=jax.ShapeDtypeStruct((BATCH, SEQ, D_MODEL), jnp.float32),
    )(x, pos_table)
